# baseline (speedup 1.0000x reference)
import functools
import jax
import jax.numpy as jnp
from jax import lax
from jax.experimental import pallas as pl
from jax.experimental.pallas import tpu as pltpu
from jax.experimental.pallas import tpu_sc as plsc

N = 10000
NP = 10240
E = 320000
D = 128
B = 4096
NEG_SLOPE = 0.01

NC, NS = 2, 16
NW = NC * NS
CH = 128
NCHUNK = 80
EPW = CH * NCHUNK
EP = NW * EPW
RPT = NP // NS

def _sc_mesh():
    return plsc.VectorSubcoreMesh(
        core_axis_name="c", subcore_axis_name="s", num_cores=NC, num_subcores=NS)


def _deg_body(dst_hbm, ones_hbm, z_hbm, out_hbm, idx_v, ones_v, acc_sh, sem):
    cid = lax.axis_index("c")
    sid = lax.axis_index("s")
    wid = sid * NC + cid
    pltpu.sync_copy(z_hbm.at[pl.ds(sid * RPT, RPT)], acc_sh.at[pl.ds(sid * RPT, RPT)])
    pltpu.sync_copy(ones_hbm, ones_v)
    pltpu.sync_copy(dst_hbm.at[wid], idx_v)
    plsc.subcore_barrier()

    def chunk(c8, carry):
        for j in range(8):
            pltpu.async_copy(ones_v, acc_sh.at[idx_v.at[c8 * 8 + j]], sem, add=True)
        for j in range(8):
            pltpu.make_async_copy(ones_v, acc_sh.at[idx_v.at[c8 * 8 + j]], sem).wait()
        return carry

    lax.fori_loop(0, NCHUNK // 8, chunk, 0)
    plsc.subcore_barrier()
    pltpu.sync_copy(acc_sh.at[pl.ds(sid * RPT, RPT)],
                    out_hbm.at[pl.ds(cid * NP + sid * RPT, RPT)])


@functools.cache
def _deg_call():
  return pl.kernel(
    _deg_body,
    out_type=jax.ShapeDtypeStruct((NC * NP, D), jnp.float32),
    mesh=_sc_mesh(),
    scratch_types=[
        pltpu.VMEM((NCHUNK, CH), jnp.int32),
        pltpu.VMEM((CH, D), jnp.float32),
        pltpu.VMEM_SHARED((NP, D), jnp.float32),
        pltpu.SemaphoreType.DMA,
    ],
  )


def _spmv_body(hn_hbm, src_hbm, dst_hbm, z_hbm, out_hbm,
               idx_sa, idx_sb, idx_da, idx_db, rows_a, rows_b,
               acc_sh, sem_a, sem_b):
    cid = lax.axis_index("c")
    sid = lax.axis_index("s")
    wid = sid * NC + cid
    pltpu.sync_copy(z_hbm.at[pl.ds(sid * RPT, RPT)], acc_sh.at[pl.ds(sid * RPT, RPT)])
    plsc.subcore_barrier()

    pltpu.sync_copy(src_hbm.at[wid, 0], idx_sa)
    pltpu.async_copy(hn_hbm.at[idx_sa], rows_a, sem_a)

    def chunk(c2, carry):
        c = 2 * c2
        pltpu.sync_copy(src_hbm.at[wid, c + 1], idx_sb)
        pltpu.async_copy(hn_hbm.at[idx_sb], rows_b, sem_b)
        pltpu.make_async_copy(hn_hbm.at[idx_sa], rows_a, sem_a).wait()
        pltpu.sync_copy(dst_hbm.at[wid, c], idx_da)
        pltpu.sync_copy(rows_a, acc_sh.at[idx_da], add=True)
        nxt = jnp.where(c2 == NCHUNK // 2 - 1, 0, c + 2)
        pltpu.sync_copy(src_hbm.at[wid, nxt], idx_sa)
        pltpu.async_copy(hn_hbm.at[idx_sa], rows_a, sem_a)
        pltpu.make_async_copy(hn_hbm.at[idx_sb], rows_b, sem_b).wait()
        pltpu.sync_copy(dst_hbm.at[wid, c + 1], idx_db)
        pltpu.sync_copy(rows_b, acc_sh.at[idx_db], add=True)
        return carry

    lax.fori_loop(0, NCHUNK // 2, chunk, 0)
    pltpu.make_async_copy(hn_hbm.at[idx_sa], rows_a, sem_a).wait()
    plsc.subcore_barrier()
    pltpu.sync_copy(acc_sh.at[pl.ds(sid * RPT, RPT)],
                    out_hbm.at[pl.ds(cid * NP + sid * RPT, RPT)])


@functools.cache
def _spmv_call():
  return pl.kernel(
    _spmv_body,
    out_type=jax.ShapeDtypeStruct((NC * NP, D), jnp.float32),
    mesh=_sc_mesh(),
    scratch_types=[
        pltpu.VMEM((CH,), jnp.int32),
        pltpu.VMEM((CH,), jnp.int32),
        pltpu.VMEM((CH,), jnp.int32),
        pltpu.VMEM((CH,), jnp.int32),
        pltpu.VMEM((CH, D), jnp.float32),
        pltpu.VMEM((CH, D), jnp.float32),
        pltpu.VMEM_SHARED((NP, D), jnp.float32),
        pltpu.SemaphoreType.DMA,
        pltpu.SemaphoreType.DMA,
    ],
  )


PPW = B // NW


def _pairs_body(h0, h1, h2, h3, u_hbm, i_hbm,
                ou0, ou1, ou2, ou3, oi0, oi1, oi2, oi3,
                uix, iix, buf, sem):
    cid = lax.axis_index("c")
    sid = lax.axis_index("s")
    wid = sid * NC + cid
    base = wid * PPW
    pltpu.sync_copy(u_hbm.at[pl.ds(base, PPW)], uix)
    pltpu.sync_copy(i_hbm.at[pl.ds(base, PPW)], iix)
    for tbl, ou, oi in ((h0, ou0, oi0), (h1, ou1, oi1),
                        (h2, ou2, oi2), (h3, ou3, oi3)):
        pltpu.async_copy(tbl.at[uix], buf, sem).wait()
        pltpu.sync_copy(buf, ou.at[pl.ds(base, PPW)])
        pltpu.async_copy(tbl.at[iix], buf, sem).wait()
        pltpu.sync_copy(buf, oi.at[pl.ds(base, PPW)])


@functools.cache
def _pairs_call():
  return pl.kernel(
    _pairs_body,
    out_type=tuple(jax.ShapeDtypeStruct((B, D), jnp.float32) for _ in range(8)),
    mesh=_sc_mesh(),
    scratch_types=[
        pltpu.VMEM((PPW,), jnp.int32),
        pltpu.VMEM((PPW,), jnp.int32),
        pltpu.VMEM((PPW, D), jnp.float32),
        pltpu.SemaphoreType.DMA,
    ],
  )


def _prep_body(degp_ref, h0_ref, dinv_ref, hn_ref):
    deg = degp_ref[0:NP, 0:1] + degp_ref[NP:2 * NP, 0:1]
    dinv = lax.rsqrt(jnp.maximum(deg, 1.0))
    dinv_ref[...] = dinv
    hn_ref[...] = h0_ref[...] * dinv


def _tc_prep(degp, h0p):
    return pl.pallas_call(
        _prep_body,
        out_shape=(jax.ShapeDtypeStruct((NP, 1), jnp.float32),
                   jax.ShapeDtypeStruct((NP, D), jnp.float32)),
    )(degp, h0p)


RB = 2048


def _layer_body(h_ref, s0_ref, s1_ref, dinv_ref, w1_ref, w2_ref,
                hp_ref, hn_ref):
    dinv = dinv_ref[...]
    s = dinv * (s0_ref[...] + s1_ref[...])
    h = h_ref[...]
    a = (jnp.dot(h + s, w1_ref[...], preferred_element_type=jnp.float32)
         + jnp.dot(h * s, w2_ref[...], preferred_element_type=jnp.float32))
    hp = jnp.where(a > 0, a, NEG_SLOPE * a)
    hp_ref[...] = hp
    hn_ref[...] = dinv * hp


def _tc_layer(h, sraw, dinv, W1, W2):
    row = lambda i: (i, 0)
    nb = NP // RB
    return pl.pallas_call(
        _layer_body,
        grid=(nb,),
        in_specs=[
            pl.BlockSpec((RB, D), row),
            pl.BlockSpec((RB, D), row),
            pl.BlockSpec((RB, D), lambda i, nb=nb: (i + nb, 0)),
            pl.BlockSpec((RB, 1), row),
            pl.BlockSpec((D, D), lambda i: (0, 0)),
            pl.BlockSpec((D, D), lambda i: (0, 0)),
        ],
        out_specs=(pl.BlockSpec((RB, D), row), pl.BlockSpec((RB, D), row)),
        out_shape=(jax.ShapeDtypeStruct((NP, D), jnp.float32),
                   jax.ShapeDtypeStruct((NP, D), jnp.float32)),
    )(h, sraw, sraw, dinv, W1, W2)


DB = 512


def _dot_body(u0, u1, u2, u3, i0, i1, i2, i3, out_ref):
    acc = jnp.sum(u0[...] * i0[...], axis=1, keepdims=True)
    acc += jnp.sum(u1[...] * i1[...], axis=1, keepdims=True)
    acc += jnp.sum(u2[...] * i2[...], axis=1, keepdims=True)
    acc += jnp.sum(u3[...] * i3[...], axis=1, keepdims=True)
    out_ref[...] = acc


def _tc_dot(gathered):
    row = lambda i: (i, 0)
    return pl.pallas_call(
        _dot_body,
        grid=(B // DB,),
        in_specs=[pl.BlockSpec((DB, D), row) for _ in range(8)],
        out_specs=pl.BlockSpec((DB, 1), row),
        out_shape=jax.ShapeDtypeStruct((B, 1), jnp.float32),
    )(*gathered)


@jax.jit
def kernel(features, feature_values, edge_index, h0,
           W1_0, W2_0, W1_1, W2_1, W1_2, W2_2):
    del feature_values
    src = edge_index[0].astype(jnp.int32)
    dst = edge_index[1].astype(jnp.int32)
    pad = EP - E
    src3 = jnp.concatenate([src, jnp.zeros((pad,), jnp.int32)]).reshape(NW, NCHUNK, CH)
    dst3 = jnp.concatenate([dst, jnp.full((pad,), N, jnp.int32)]).reshape(NW, NCHUNK, CH)
    zD = jnp.zeros((NP, D), jnp.float32)
    ones = jnp.ones((CH, D), jnp.float32)
    h0p = jnp.pad(h0, ((0, NP - N), (0, 0)))

    degp = _deg_call()(dst3, ones, zD)
    dinv, hn = _tc_prep(degp, h0p)

    h = h0p
    hs = [h0p]
    for (W1, W2) in ((W1_0, W2_0), (W1_1, W2_1), (W1_2, W2_2)):
        sraw = _spmv_call()(hn, src3, dst3, zD)
        h, hn = _tc_layer(h, sraw, dinv, W1, W2)
        hs.append(h)

    users = features[:, 0].astype(jnp.int32)
    items = features[:, 1].astype(jnp.int32)
    gathered = _pairs_call()(hs[0], hs[1], hs[2], hs[3], users, items)
    return _tc_dot(gathered)[:, 0]


# speedup vs baseline: 1.0065x; 1.0065x over previous
import functools
import jax
import jax.numpy as jnp
from jax import lax
from jax.experimental import pallas as pl
from jax.experimental.pallas import tpu as pltpu
from jax.experimental.pallas import tpu_sc as plsc

N = 10000
NP = 10240
E = 320000
D = 128
B = 4096
NEG_SLOPE = 0.01

NC, NS = 2, 16
NW = NC * NS
CH = 128
NCHUNK = 80
EPW = CH * NCHUNK
EP = NW * EPW
RPT = NP // NS

def _sc_mesh():
    return plsc.VectorSubcoreMesh(
        core_axis_name="c", subcore_axis_name="s", num_cores=NC, num_subcores=NS)


def _deg_body(dst_hbm, ones_hbm, z_hbm, out_hbm, idx_v, ones_v, acc_sh, sem):
    cid = lax.axis_index("c")
    sid = lax.axis_index("s")
    wid = sid * NC + cid
    pltpu.sync_copy(z_hbm.at[pl.ds(sid * RPT, RPT)], acc_sh.at[pl.ds(sid * RPT, RPT)])
    pltpu.sync_copy(ones_hbm, ones_v)
    pltpu.sync_copy(dst_hbm.at[wid], idx_v)
    plsc.subcore_barrier()

    def chunk(c8, carry):
        for j in range(8):
            pltpu.async_copy(ones_v, acc_sh.at[idx_v.at[c8 * 8 + j]], sem, add=True)
        for j in range(8):
            pltpu.make_async_copy(ones_v, acc_sh.at[idx_v.at[c8 * 8 + j]], sem).wait()
        return carry

    lax.fori_loop(0, NCHUNK // 8, chunk, 0)
    plsc.subcore_barrier()
    pltpu.sync_copy(acc_sh.at[pl.ds(sid * RPT, RPT)],
                    out_hbm.at[pl.ds(cid * NP + sid * RPT, RPT)])


@functools.cache
def _deg_call():
  return pl.kernel(
    _deg_body,
    out_type=jax.ShapeDtypeStruct((NC * NP, D), jnp.float32),
    mesh=_sc_mesh(),
    scratch_types=[
        pltpu.VMEM((NCHUNK, CH), jnp.int32),
        pltpu.VMEM((CH, D), jnp.float32),
        pltpu.VMEM_SHARED((NP, D), jnp.float32),
        pltpu.SemaphoreType.DMA,
    ],
  )


def _spmv_body(hn_hbm, src_hbm, dst_hbm, z_hbm, out_hbm,
               dstp_v, idx_sa, idx_sb, rows_a, rows_b,
               acc_sh, sem_a, sem_b):
    cid = lax.axis_index("c")
    sid = lax.axis_index("s")
    wid = sid * NC + cid
    base = wid * EPW
    pltpu.sync_copy(z_hbm.at[pl.ds(sid * RPT, RPT)], acc_sh.at[pl.ds(sid * RPT, RPT)])
    pltpu.sync_copy(dst_hbm.at[wid], dstp_v)
    plsc.subcore_barrier()

    pltpu.sync_copy(src_hbm.at[pl.ds(base, CH)], idx_sa)
    pltpu.async_copy(hn_hbm.at[idx_sa], rows_a, sem_a)

    def chunk(c2, carry):
        c = 2 * c2
        pltpu.sync_copy(src_hbm.at[pl.ds(base + (c + 1) * CH, CH)], idx_sb)
        pltpu.async_copy(hn_hbm.at[idx_sb], rows_b, sem_b)
        pltpu.make_async_copy(hn_hbm.at[idx_sa], rows_a, sem_a).wait()
        pltpu.sync_copy(rows_a, acc_sh.at[dstp_v.at[c]], add=True)
        nxt = jnp.where(c2 == NCHUNK // 2 - 1, 0, c + 2)
        pltpu.sync_copy(src_hbm.at[pl.ds(base + nxt * CH, CH)], idx_sa)
        pltpu.async_copy(hn_hbm.at[idx_sa], rows_a, sem_a)
        pltpu.make_async_copy(hn_hbm.at[idx_sb], rows_b, sem_b).wait()
        pltpu.sync_copy(rows_b, acc_sh.at[dstp_v.at[c + 1]], add=True)
        return carry

    lax.fori_loop(0, NCHUNK // 2, chunk, 0)
    pltpu.make_async_copy(hn_hbm.at[idx_sa], rows_a, sem_a).wait()
    plsc.subcore_barrier()
    pltpu.sync_copy(acc_sh.at[pl.ds(sid * RPT, RPT)],
                    out_hbm.at[pl.ds(cid * NP + sid * RPT, RPT)])


@functools.cache
def _spmv_call():
  return pl.kernel(
    _spmv_body,
    out_type=jax.ShapeDtypeStruct((NC * NP, D), jnp.float32),
    mesh=_sc_mesh(),
    scratch_types=[
        pltpu.VMEM((NCHUNK, CH), jnp.int32),
        pltpu.VMEM((CH,), jnp.int32),
        pltpu.VMEM((CH,), jnp.int32),
        pltpu.VMEM((CH, D), jnp.float32),
        pltpu.VMEM((CH, D), jnp.float32),
        pltpu.VMEM_SHARED((NP, D), jnp.float32),
        pltpu.SemaphoreType.DMA,
        pltpu.SemaphoreType.DMA,
    ],
  )


PPW = B // NW


def _pairs_body(h0, h1, h2, h3, u_hbm, i_hbm,
                ou0, ou1, ou2, ou3, oi0, oi1, oi2, oi3,
                uix, iix, buf, sem):
    cid = lax.axis_index("c")
    sid = lax.axis_index("s")
    wid = sid * NC + cid
    base = wid * PPW
    pltpu.sync_copy(u_hbm.at[pl.ds(base, PPW)], uix)
    pltpu.sync_copy(i_hbm.at[pl.ds(base, PPW)], iix)
    for tbl, ou, oi in ((h0, ou0, oi0), (h1, ou1, oi1),
                        (h2, ou2, oi2), (h3, ou3, oi3)):
        pltpu.async_copy(tbl.at[uix], buf, sem).wait()
        pltpu.sync_copy(buf, ou.at[pl.ds(base, PPW)])
        pltpu.async_copy(tbl.at[iix], buf, sem).wait()
        pltpu.sync_copy(buf, oi.at[pl.ds(base, PPW)])


@functools.cache
def _pairs_call():
  return pl.kernel(
    _pairs_body,
    out_type=tuple(jax.ShapeDtypeStruct((B, D), jnp.float32) for _ in range(8)),
    mesh=_sc_mesh(),
    scratch_types=[
        pltpu.VMEM((PPW,), jnp.int32),
        pltpu.VMEM((PPW,), jnp.int32),
        pltpu.VMEM((PPW, D), jnp.float32),
        pltpu.SemaphoreType.DMA,
    ],
  )


def _prep_body(degp_ref, h0_ref, dinv_ref, hn_ref):
    deg = degp_ref[0:NP, 0:1] + degp_ref[NP:2 * NP, 0:1]
    dinv = lax.rsqrt(jnp.maximum(deg, 1.0))
    dinv_ref[...] = dinv
    hn_ref[...] = h0_ref[...] * dinv


def _tc_prep(degp, h0p):
    return pl.pallas_call(
        _prep_body,
        out_shape=(jax.ShapeDtypeStruct((NP, 1), jnp.float32),
                   jax.ShapeDtypeStruct((NP, D), jnp.float32)),
    )(degp, h0p)


RB = 2048


def _layer_body(h_ref, s0_ref, s1_ref, dinv_ref, w1_ref, w2_ref,
                hp_ref, hn_ref):
    dinv = dinv_ref[...]
    s = dinv * (s0_ref[...] + s1_ref[...])
    h = h_ref[...]
    a = (jnp.dot(h + s, w1_ref[...], preferred_element_type=jnp.float32)
         + jnp.dot(h * s, w2_ref[...], preferred_element_type=jnp.float32))
    hp = jnp.where(a > 0, a, NEG_SLOPE * a)
    hp_ref[...] = hp
    hn_ref[...] = dinv * hp


def _tc_layer(h, sraw, dinv, W1, W2):
    row = lambda i: (i, 0)
    nb = NP // RB
    return pl.pallas_call(
        _layer_body,
        grid=(nb,),
        in_specs=[
            pl.BlockSpec((RB, D), row),
            pl.BlockSpec((RB, D), row),
            pl.BlockSpec((RB, D), lambda i, nb=nb: (i + nb, 0)),
            pl.BlockSpec((RB, 1), row),
            pl.BlockSpec((D, D), lambda i: (0, 0)),
            pl.BlockSpec((D, D), lambda i: (0, 0)),
        ],
        out_specs=(pl.BlockSpec((RB, D), row), pl.BlockSpec((RB, D), row)),
        out_shape=(jax.ShapeDtypeStruct((NP, D), jnp.float32),
                   jax.ShapeDtypeStruct((NP, D), jnp.float32)),
    )(h, sraw, sraw, dinv, W1, W2)


DB = 512


def _dot_body(u0, u1, u2, u3, i0, i1, i2, i3, out_ref):
    acc = jnp.sum(u0[...] * i0[...], axis=1, keepdims=True)
    acc += jnp.sum(u1[...] * i1[...], axis=1, keepdims=True)
    acc += jnp.sum(u2[...] * i2[...], axis=1, keepdims=True)
    acc += jnp.sum(u3[...] * i3[...], axis=1, keepdims=True)
    out_ref[...] = acc


def _tc_dot(gathered):
    row = lambda i: (i, 0)
    return pl.pallas_call(
        _dot_body,
        grid=(B // DB,),
        in_specs=[pl.BlockSpec((DB, D), row) for _ in range(8)],
        out_specs=pl.BlockSpec((DB, 1), row),
        out_shape=jax.ShapeDtypeStruct((B, 1), jnp.float32),
    )(*gathered)


@jax.jit
def kernel(features, feature_values, edge_index, h0,
           W1_0, W2_0, W1_1, W2_1, W1_2, W2_2):
    del feature_values
    src = edge_index[0].astype(jnp.int32)
    dst = edge_index[1].astype(jnp.int32)
    pad = EP - E
    srcp = jnp.concatenate([src, jnp.zeros((pad,), jnp.int32)])
    dst3 = jnp.concatenate([dst, jnp.full((pad,), N, jnp.int32)]).reshape(NW, NCHUNK, CH)
    zD = jnp.zeros((NP, D), jnp.float32)
    ones = jnp.ones((CH, D), jnp.float32)
    h0p = jnp.pad(h0, ((0, NP - N), (0, 0)))

    degp = _deg_call()(dst3, ones, zD)
    dinv, hn = _tc_prep(degp, h0p)

    h = h0p
    hs = [h0p]
    for (W1, W2) in ((W1_0, W2_0), (W1_1, W2_1), (W1_2, W2_2)):
        sraw = _spmv_call()(hn, srcp, dst3, zD)
        h, hn = _tc_layer(h, sraw, dinv, W1, W2)
        hs.append(h)

    users = features[:, 0].astype(jnp.int32)
    items = features[:, 1].astype(jnp.int32)
    gathered = _pairs_call()(hs[0], hs[1], hs[2], hs[3], users, items)
    return _tc_dot(gathered)[:, 0]


# speedup vs baseline: 1.0504x; 1.0436x over previous
import functools
import jax
import jax.numpy as jnp
from jax import lax
from jax.experimental import pallas as pl
from jax.experimental.pallas import tpu as pltpu
from jax.experimental.pallas import tpu_sc as plsc

N = 10000
NP = 10112
E = 320000
D = 128
B = 4096
NEG_SLOPE = 0.01

NC, NS = 2, 16
NW = NC * NS
CH = 64
NCHUNK = 160
EPW = CH * NCHUNK
EP = NW * EPW
RPT = NP // NS

def _sc_mesh():
    return plsc.VectorSubcoreMesh(
        core_axis_name="c", subcore_axis_name="s", num_cores=NC, num_subcores=NS)


def _deg_body(dst_hbm, ones_hbm, z_hbm, out_hbm, idx_v, ones_v, acc_sh, sem):
    cid = lax.axis_index("c")
    sid = lax.axis_index("s")
    wid = sid * NC + cid
    pltpu.sync_copy(z_hbm.at[pl.ds(sid * RPT, RPT)], acc_sh.at[pl.ds(sid * RPT, RPT)])
    pltpu.sync_copy(ones_hbm, ones_v)
    pltpu.sync_copy(dst_hbm.at[wid], idx_v)
    plsc.subcore_barrier()

    def chunk(c8, carry):
        for j in range(8):
            pltpu.async_copy(ones_v, acc_sh.at[idx_v.at[c8 * 8 + j]], sem, add=True)
        for j in range(8):
            pltpu.make_async_copy(ones_v, acc_sh.at[idx_v.at[c8 * 8 + j]], sem).wait()
        return carry

    lax.fori_loop(0, NCHUNK // 8, chunk, 0)
    plsc.subcore_barrier()
    pltpu.sync_copy(acc_sh.at[pl.ds(sid * RPT, RPT)],
                    out_hbm.at[pl.ds(cid * NP + sid * RPT, RPT)])


@functools.cache
def _deg_call():
  return pl.kernel(
    _deg_body,
    out_type=jax.ShapeDtypeStruct((NC * NP, D), jnp.float32),
    mesh=_sc_mesh(),
    scratch_types=[
        pltpu.VMEM((NCHUNK, CH), jnp.int32),
        pltpu.VMEM((CH, D), jnp.float32),
        pltpu.VMEM_SHARED((NP, D), jnp.float32),
        pltpu.SemaphoreType.DMA,
    ],
  )


def _spmv_body(hn_hbm, src_hbm, dst_hbm, z_hbm, out_hbm,
               isrc0, isrc1, isrc2, isrc3, idst0, idst1, idst2, idst3,
               rows0, rows1, rows2, rows3, acc_sh,
               sg0, sg1, sg2, sg3, ss0, ss1, ss2, ss3,
               sis0, sis1, sis2, sis3, sid0, sid1, sid2, sid3):
    cid = lax.axis_index("c")
    sid = lax.axis_index("s")
    wid = sid * NC + cid
    base = wid * EPW
    rows = (rows0, rows1, rows2, rows3)
    isrc = (isrc0, isrc1, isrc2, isrc3)
    idst = (idst0, idst1, idst2, idst3)
    sg = (sg0, sg1, sg2, sg3)
    ss = (ss0, ss1, ss2, ss3)
    sis = (sis0, sis1, sis2, sis3)
    sidm = (sid0, sid1, sid2, sid3)
    pltpu.sync_copy(z_hbm.at[pl.ds(sid * RPT, RPT)], acc_sh.at[pl.ds(sid * RPT, RPT)])
    plsc.subcore_barrier()

    for j in (0, 1):
        pltpu.sync_copy(src_hbm.at[pl.ds(base + j * CH, CH)], isrc[j])
        pltpu.sync_copy(dst_hbm.at[pl.ds(base + j * CH, CH)], idst[j])
        pltpu.async_copy(hn_hbm.at[isrc[j]], rows[j], sg[j])

    def block(c2, carry):
        for j in range(4):
            c = 4 * c2 + j
            b = j
            b2 = (j + 2) % 4
            if j < 2:
                nxt = c + 2
            else:
                nxt = jnp.where(c2 == NCHUNK // 4 - 1, j - 2, c + 2)
            pltpu.async_copy(src_hbm.at[pl.ds(base + nxt * CH, CH)], isrc[b2], sis[b2])
            pltpu.make_async_copy(hn_hbm.at[isrc[b]], rows[b], sg[b]).wait()
            if j < 2:
                @pl.when(c2 > 0)
                def _():
                    pltpu.make_async_copy(dst_hbm.at[pl.ds(base, CH)], idst[b], sidm[b]).wait()
            else:
                pltpu.make_async_copy(dst_hbm.at[pl.ds(base, CH)], idst[b], sidm[b]).wait()
            pltpu.async_copy(rows[b], acc_sh.at[idst[b]], ss[b], add=True)
            if j < 2:
                @pl.when(c2 > 0)
                def _():
                    pltpu.make_async_copy(rows[b2], acc_sh.at[idst[b2]], ss[b2]).wait()
            else:
                pltpu.make_async_copy(rows[b2], acc_sh.at[idst[b2]], ss[b2]).wait()
            pltpu.async_copy(dst_hbm.at[pl.ds(base + nxt * CH, CH)], idst[b2], sidm[b2])
            pltpu.make_async_copy(src_hbm.at[pl.ds(base, CH)], isrc[b2], sis[b2]).wait()
            pltpu.async_copy(hn_hbm.at[isrc[b2]], rows[b2], sg[b2])
        return carry

    lax.fori_loop(0, NCHUNK // 4, block, 0)
    for j in (0, 1):
        pltpu.make_async_copy(hn_hbm.at[isrc[j]], rows[j], sg[j]).wait()
        pltpu.make_async_copy(dst_hbm.at[pl.ds(base, CH)], idst[j], sidm[j]).wait()
    pltpu.make_async_copy(rows[2], acc_sh.at[idst[2]], ss[2]).wait()
    pltpu.make_async_copy(rows[3], acc_sh.at[idst[3]], ss[3]).wait()
    plsc.subcore_barrier()
    pltpu.sync_copy(acc_sh.at[pl.ds(sid * RPT, RPT)],
                    out_hbm.at[pl.ds(cid * NP + sid * RPT, RPT)])


@functools.cache
def _spmv_call():
  return pl.kernel(
    _spmv_body,
    out_type=jax.ShapeDtypeStruct((NC * NP, D), jnp.float32),
    mesh=_sc_mesh(),
    scratch_types=(
        [pltpu.VMEM((CH,), jnp.int32) for _ in range(8)]
        + [pltpu.VMEM((CH, D), jnp.float32) for _ in range(4)]
        + [pltpu.VMEM_SHARED((NP, D), jnp.float32)]
        + [pltpu.SemaphoreType.DMA for _ in range(16)]
    ),
  )


PPW = B // NW


def _pairs_body(h0, h1, h2, h3, u_hbm, i_hbm,
                ou0, ou1, ou2, ou3, oi0, oi1, oi2, oi3,
                uix, iix, buf, sem):
    cid = lax.axis_index("c")
    sid = lax.axis_index("s")
    wid = sid * NC + cid
    base = wid * PPW
    pltpu.sync_copy(u_hbm.at[pl.ds(base, PPW)], uix)
    pltpu.sync_copy(i_hbm.at[pl.ds(base, PPW)], iix)
    for tbl, ou, oi in ((h0, ou0, oi0), (h1, ou1, oi1),
                        (h2, ou2, oi2), (h3, ou3, oi3)):
        pltpu.async_copy(tbl.at[uix], buf, sem).wait()
        pltpu.sync_copy(buf, ou.at[pl.ds(base, PPW)])
        pltpu.async_copy(tbl.at[iix], buf, sem).wait()
        pltpu.sync_copy(buf, oi.at[pl.ds(base, PPW)])


@functools.cache
def _pairs_call():
  return pl.kernel(
    _pairs_body,
    out_type=tuple(jax.ShapeDtypeStruct((B, D), jnp.float32) for _ in range(8)),
    mesh=_sc_mesh(),
    scratch_types=[
        pltpu.VMEM((PPW,), jnp.int32),
        pltpu.VMEM((PPW,), jnp.int32),
        pltpu.VMEM((PPW, D), jnp.float32),
        pltpu.SemaphoreType.DMA,
    ],
  )


def _prep_body(degp_ref, h0_ref, dinv_ref, hn_ref):
    deg = degp_ref[0:NP, 0:1] + degp_ref[NP:2 * NP, 0:1]
    dinv = lax.rsqrt(jnp.maximum(deg, 1.0))
    dinv_ref[...] = dinv
    hn_ref[...] = h0_ref[...] * dinv


def _tc_prep(degp, h0p):
    return pl.pallas_call(
        _prep_body,
        out_shape=(jax.ShapeDtypeStruct((NP, 1), jnp.float32),
                   jax.ShapeDtypeStruct((NP, D), jnp.float32)),
    )(degp, h0p)


RB = 1264


def _layer_body(h_ref, s0_ref, s1_ref, dinv_ref, w1_ref, w2_ref,
                hp_ref, hn_ref):
    dinv = dinv_ref[...]
    s = dinv * (s0_ref[...] + s1_ref[...])
    h = h_ref[...]
    a = (jnp.dot(h + s, w1_ref[...], preferred_element_type=jnp.float32)
         + jnp.dot(h * s, w2_ref[...], preferred_element_type=jnp.float32))
    hp = jnp.where(a > 0, a, NEG_SLOPE * a)
    hp_ref[...] = hp
    hn_ref[...] = dinv * hp


def _tc_layer(h, sraw, dinv, W1, W2):
    row = lambda i: (i, 0)
    nb = NP // RB
    return pl.pallas_call(
        _layer_body,
        grid=(nb,),
        in_specs=[
            pl.BlockSpec((RB, D), row),
            pl.BlockSpec((RB, D), row),
            pl.BlockSpec((RB, D), lambda i, nb=nb: (i + nb, 0)),
            pl.BlockSpec((RB, 1), row),
            pl.BlockSpec((D, D), lambda i: (0, 0)),
            pl.BlockSpec((D, D), lambda i: (0, 0)),
        ],
        out_specs=(pl.BlockSpec((RB, D), row), pl.BlockSpec((RB, D), row)),
        out_shape=(jax.ShapeDtypeStruct((NP, D), jnp.float32),
                   jax.ShapeDtypeStruct((NP, D), jnp.float32)),
    )(h, sraw, sraw, dinv, W1, W2)


DB = 512


def _dot_body(u0, u1, u2, u3, i0, i1, i2, i3, out_ref):
    acc = jnp.sum(u0[...] * i0[...], axis=1, keepdims=True)
    acc += jnp.sum(u1[...] * i1[...], axis=1, keepdims=True)
    acc += jnp.sum(u2[...] * i2[...], axis=1, keepdims=True)
    acc += jnp.sum(u3[...] * i3[...], axis=1, keepdims=True)
    out_ref[...] = acc


def _tc_dot(gathered):
    row = lambda i: (i, 0)
    return pl.pallas_call(
        _dot_body,
        grid=(B // DB,),
        in_specs=[pl.BlockSpec((DB, D), row) for _ in range(8)],
        out_specs=pl.BlockSpec((DB, 1), row),
        out_shape=jax.ShapeDtypeStruct((B, 1), jnp.float32),
    )(*gathered)


@jax.jit
def kernel(features, feature_values, edge_index, h0,
           W1_0, W2_0, W1_1, W2_1, W1_2, W2_2):
    del feature_values
    src = edge_index[0].astype(jnp.int32)
    dst = edge_index[1].astype(jnp.int32)
    pad = EP - E
    srcp = jnp.concatenate([src, jnp.zeros((pad,), jnp.int32)])
    dstp = jnp.concatenate([dst, jnp.full((pad,), N, jnp.int32)])
    dst3 = dstp.reshape(NW, NCHUNK, CH)
    zD = jnp.zeros((NP, D), jnp.float32)
    ones = jnp.ones((CH, D), jnp.float32)
    h0p = jnp.pad(h0, ((0, NP - N), (0, 0)))

    degp = _deg_call()(dst3, ones, zD)
    dinv, hn = _tc_prep(degp, h0p)

    h = h0p
    hs = [h0p]
    for (W1, W2) in ((W1_0, W2_0), (W1_1, W2_1), (W1_2, W2_2)):
        sraw = _spmv_call()(hn, srcp, dstp, zD)
        h, hn = _tc_layer(h, sraw, dinv, W1, W2)
        hs.append(h)

    users = features[:, 0].astype(jnp.int32)
    items = features[:, 1].astype(jnp.int32)
    gathered = _pairs_call()(hs[0], hs[1], hs[2], hs[3], users, items)
    return _tc_dot(gathered)[:, 0]


# speedup vs baseline: 3.0314x; 2.8860x over previous
import functools
import jax
import jax.numpy as jnp
from jax import lax
from jax.experimental import pallas as pl
from jax.experimental.pallas import tpu as pltpu
from jax.experimental.pallas import tpu_sc as plsc

N = 10000
NP = 10112
E = 320000
D = 128
B = 4096
NEG_SLOPE = 0.01

NC, NS = 2, 16
NW = NC * NS
CH = 64
NCHUNK = 160
EPW = CH * NCHUNK
EP = NW * EPW
RPT = NP // NS

def _sc_mesh():
    return plsc.VectorSubcoreMesh(
        core_axis_name="c", subcore_axis_name="s", num_cores=NC, num_subcores=NS)


def _deg_body(dst_hbm, ones_hbm, z_hbm, out_hbm, idx_v, ones_v, acc_sh, sem):
    cid = lax.axis_index("c")
    sid = lax.axis_index("s")
    wid = sid * NC + cid
    pltpu.sync_copy(z_hbm.at[pl.ds(sid * RPT, RPT)], acc_sh.at[pl.ds(sid * RPT, RPT)])
    pltpu.sync_copy(ones_hbm, ones_v)
    pltpu.sync_copy(dst_hbm.at[wid], idx_v)
    plsc.subcore_barrier()

    def chunk(c8, carry):
        for j in range(8):
            pltpu.async_copy(ones_v, acc_sh.at[idx_v.at[c8 * 8 + j]], sem, add=True)
        for j in range(8):
            pltpu.make_async_copy(ones_v, acc_sh.at[idx_v.at[c8 * 8 + j]], sem).wait()
        return carry

    lax.fori_loop(0, NCHUNK // 8, chunk, 0)
    plsc.subcore_barrier()
    pltpu.sync_copy(acc_sh.at[pl.ds(sid * RPT, RPT)],
                    out_hbm.at[pl.ds(cid * NP + sid * RPT, RPT)])


@functools.cache
def _deg_call():
  return pl.kernel(
    _deg_body,
    out_type=jax.ShapeDtypeStruct((NC * NP, D), jnp.float32),
    mesh=_sc_mesh(),
    scratch_types=[
        pltpu.VMEM((NCHUNK, CH), jnp.int32),
        pltpu.VMEM((CH, D), jnp.float32),
        pltpu.VMEM_SHARED((NP, D), jnp.float32),
        pltpu.SemaphoreType.DMA,
    ],
  )


def _spmv_body(hn_hbm, src_hbm, dst_hbm, z_hbm, out_hbm,
               isrc0, isrc1, isrc2, isrc3, idst0, idst1, idst2, idst3,
               rows0, rows1, rows2, rows3, acc_sh,
               sg0, sg1, sg2, sg3, ss0, ss1, ss2, ss3,
               sis0, sis1, sis2, sis3, sid0, sid1, sid2, sid3):
    cid = lax.axis_index("c")
    sid = lax.axis_index("s")
    wid = sid * NC + cid
    base = wid * EPW
    rows = (rows0, rows1, rows2, rows3)
    isrc = (isrc0, isrc1, isrc2, isrc3)
    idst = (idst0, idst1, idst2, idst3)
    sg = (sg0, sg1, sg2, sg3)
    ss = (ss0, ss1, ss2, ss3)
    sis = (sis0, sis1, sis2, sis3)
    sidm = (sid0, sid1, sid2, sid3)
    pltpu.sync_copy(z_hbm.at[pl.ds(sid * RPT, RPT)], acc_sh.at[pl.ds(sid * RPT, RPT)])
    plsc.subcore_barrier()

    for j in (0, 1):
        pltpu.sync_copy(src_hbm.at[pl.ds(base + j * CH, CH)], isrc[j])
        pltpu.sync_copy(dst_hbm.at[pl.ds(base + j * CH, CH)], idst[j])
        pltpu.async_copy(hn_hbm.at[isrc[j]], rows[j], sg[j])

    def block(c2, carry):
        for j in range(4):
            c = 4 * c2 + j
            b = j
            b2 = (j + 2) % 4
            if j < 2:
                nxt = c + 2
            else:
                nxt = jnp.where(c2 == NCHUNK // 4 - 1, j - 2, c + 2)
            pltpu.async_copy(src_hbm.at[pl.ds(base + nxt * CH, CH)], isrc[b2], sis[b2])
            pltpu.make_async_copy(hn_hbm.at[isrc[b]], rows[b], sg[b]).wait()
            if j < 2:
                @pl.when(c2 > 0)
                def _():
                    pltpu.make_async_copy(dst_hbm.at[pl.ds(base, CH)], idst[b], sidm[b]).wait()
            else:
                pltpu.make_async_copy(dst_hbm.at[pl.ds(base, CH)], idst[b], sidm[b]).wait()
            pltpu.async_copy(rows[b], acc_sh.at[idst[b]], ss[b], add=True)
            if j < 2:
                @pl.when(c2 > 0)
                def _():
                    pltpu.make_async_copy(rows[b2], acc_sh.at[idst[b2]], ss[b2]).wait()
            else:
                pltpu.make_async_copy(rows[b2], acc_sh.at[idst[b2]], ss[b2]).wait()
            pltpu.async_copy(dst_hbm.at[pl.ds(base + nxt * CH, CH)], idst[b2], sidm[b2])
            pltpu.make_async_copy(src_hbm.at[pl.ds(base, CH)], isrc[b2], sis[b2]).wait()
            pltpu.async_copy(hn_hbm.at[isrc[b2]], rows[b2], sg[b2])
        return carry

    lax.fori_loop(0, NCHUNK // 4, block, 0)
    for j in (0, 1):
        pltpu.make_async_copy(hn_hbm.at[isrc[j]], rows[j], sg[j]).wait()
        pltpu.make_async_copy(dst_hbm.at[pl.ds(base, CH)], idst[j], sidm[j]).wait()
    pltpu.make_async_copy(rows[2], acc_sh.at[idst[2]], ss[2]).wait()
    pltpu.make_async_copy(rows[3], acc_sh.at[idst[3]], ss[3]).wait()
    plsc.subcore_barrier()
    pltpu.sync_copy(acc_sh.at[pl.ds(sid * RPT, RPT)],
                    out_hbm.at[pl.ds(cid * NP + sid * RPT, RPT)])


@functools.cache
def _spmv_call():
  return pl.kernel(
    _spmv_body,
    out_type=jax.ShapeDtypeStruct((NC * NP, D), jnp.float32),
    mesh=_sc_mesh(),
    scratch_types=(
        [pltpu.VMEM((CH,), jnp.int32) for _ in range(8)]
        + [pltpu.VMEM((CH, D), jnp.float32) for _ in range(4)]
        + [pltpu.VMEM_SHARED((NP, D), jnp.float32)]
        + [pltpu.SemaphoreType.DMA for _ in range(16)]
    ),
  )


PPW = B // NW


def _pairs_body(h0, h1, h2, h3, u_hbm, i_hbm,
                ou0, ou1, ou2, ou3, oi0, oi1, oi2, oi3,
                uix, iix, buf, sem):
    cid = lax.axis_index("c")
    sid = lax.axis_index("s")
    wid = sid * NC + cid
    base = wid * PPW
    pltpu.sync_copy(u_hbm.at[pl.ds(base, PPW)], uix)
    pltpu.sync_copy(i_hbm.at[pl.ds(base, PPW)], iix)
    for tbl, ou, oi in ((h0, ou0, oi0), (h1, ou1, oi1),
                        (h2, ou2, oi2), (h3, ou3, oi3)):
        pltpu.async_copy(tbl.at[uix], buf, sem).wait()
        pltpu.sync_copy(buf, ou.at[pl.ds(base, PPW)])
        pltpu.async_copy(tbl.at[iix], buf, sem).wait()
        pltpu.sync_copy(buf, oi.at[pl.ds(base, PPW)])


@functools.cache
def _pairs_call():
  return pl.kernel(
    _pairs_body,
    out_type=tuple(jax.ShapeDtypeStruct((B, D), jnp.float32) for _ in range(8)),
    mesh=_sc_mesh(),
    scratch_types=[
        pltpu.VMEM((PPW,), jnp.int32),
        pltpu.VMEM((PPW,), jnp.int32),
        pltpu.VMEM((PPW, D), jnp.float32),
        pltpu.SemaphoreType.DMA,
    ],
  )


def _prep_body(degp_ref, h0_ref, dinv_ref, hn_ref):
    deg = degp_ref[0:NP, 0:1] + degp_ref[NP:2 * NP, 0:1]
    dinv = lax.rsqrt(jnp.maximum(deg, 1.0))
    dinv_ref[...] = dinv
    hn_ref[...] = h0_ref[...] * dinv


def _tc_prep(degp, h0p):
    return pl.pallas_call(
        _prep_body,
        out_shape=(jax.ShapeDtypeStruct((NP, 1), jnp.float32),
                   jax.ShapeDtypeStruct((NP, D), jnp.float32)),
    )(degp, h0p)


RB = 1264


def _layer_body(h_ref, s0_ref, s1_ref, dinv_ref, w1_ref, w2_ref,
                hp_ref, hn_ref):
    dinv = dinv_ref[...]
    s = dinv * (s0_ref[...] + s1_ref[...])
    h = h_ref[...]
    a = (jnp.dot(h + s, w1_ref[...], preferred_element_type=jnp.float32)
         + jnp.dot(h * s, w2_ref[...], preferred_element_type=jnp.float32))
    hp = jnp.where(a > 0, a, NEG_SLOPE * a)
    hp_ref[...] = hp
    hn_ref[...] = dinv * hp


def _tc_layer(h, sraw, dinv, W1, W2):
    row = lambda i: (i, 0)
    nb = NP // RB
    return pl.pallas_call(
        _layer_body,
        grid=(nb,),
        in_specs=[
            pl.BlockSpec((RB, D), row),
            pl.BlockSpec((RB, D), row),
            pl.BlockSpec((RB, D), lambda i, nb=nb: (i + nb, 0)),
            pl.BlockSpec((RB, 1), row),
            pl.BlockSpec((D, D), lambda i: (0, 0)),
            pl.BlockSpec((D, D), lambda i: (0, 0)),
        ],
        out_specs=(pl.BlockSpec((RB, D), row), pl.BlockSpec((RB, D), row)),
        out_shape=(jax.ShapeDtypeStruct((NP, D), jnp.float32),
                   jax.ShapeDtypeStruct((NP, D), jnp.float32)),
    )(h, sraw, sraw, dinv, W1, W2)


DB = 512


def _dot_body(u0, u1, u2, u3, i0, i1, i2, i3, out_ref):
    acc = jnp.sum(u0[...] * i0[...], axis=1, keepdims=True)
    acc += jnp.sum(u1[...] * i1[...], axis=1, keepdims=True)
    acc += jnp.sum(u2[...] * i2[...], axis=1, keepdims=True)
    acc += jnp.sum(u3[...] * i3[...], axis=1, keepdims=True)
    out_ref[...] = acc


def _tc_dot(gathered):
    row = lambda i: (i, 0)
    return pl.pallas_call(
        _dot_body,
        grid=(B // DB,),
        in_specs=[pl.BlockSpec((DB, D), row) for _ in range(8)],
        out_specs=pl.BlockSpec((DB, 1), row),
        out_shape=jax.ShapeDtypeStruct((B, 1), jnp.float32),
    )(*gathered)


@jax.jit
def kernel(features, feature_values, edge_index, h0,
           W1_0, W2_0, W1_1, W2_1, W1_2, W2_2):
    del feature_values
    src = edge_index[0].astype(jnp.int32)
    dst = edge_index[1].astype(jnp.int32)
    pad = EP - E
    pad_dst = N + (jnp.arange(pad, dtype=jnp.int32) % (NP - N))
    pad_src = jnp.arange(pad, dtype=jnp.int32) % N
    srcp = jnp.concatenate([src, pad_src])
    dstp = jnp.concatenate([dst, pad_dst])
    dst3 = dstp.reshape(NW, NCHUNK, CH)
    zD = jnp.zeros((NP, D), jnp.float32)
    ones = jnp.ones((CH, D), jnp.float32)
    h0p = jnp.pad(h0, ((0, NP - N), (0, 0)))

    degp = _deg_call()(dst3, ones, zD)
    dinv, hn = _tc_prep(degp, h0p)

    h = h0p
    hs = [h0p]
    for (W1, W2) in ((W1_0, W2_0), (W1_1, W2_1), (W1_2, W2_2)):
        sraw = _spmv_call()(hn, srcp, dstp, zD)
        h, hn = _tc_layer(h, sraw, dinv, W1, W2)
        hs.append(h)

    users = features[:, 0].astype(jnp.int32)
    items = features[:, 1].astype(jnp.int32)
    gathered = _pairs_call()(hs[0], hs[1], hs[2], hs[3], users, items)
    return _tc_dot(gathered)[:, 0]


# speedup vs baseline: 3.1692x; 1.0455x over previous
import functools
import jax
import jax.numpy as jnp
from jax import lax
from jax.experimental import pallas as pl
from jax.experimental.pallas import tpu as pltpu
from jax.experimental.pallas import tpu_sc as plsc

N = 10000
NP = 10112
E = 320000
D = 128
B = 4096
NEG_SLOPE = 0.01

NC, NS = 2, 16
NW = NC * NS
CH = 80
NCHUNK = 128
EPW = CH * NCHUNK
EP = NW * EPW
RPT = NP // NS

def _sc_mesh():
    return plsc.VectorSubcoreMesh(
        core_axis_name="c", subcore_axis_name="s", num_cores=NC, num_subcores=NS)


def _deg_body(dst_hbm, ones_hbm, z_hbm, out_hbm, idx_v, ones_v, acc_sh, sem):
    cid = lax.axis_index("c")
    sid = lax.axis_index("s")
    wid = sid * NC + cid
    pltpu.sync_copy(z_hbm.at[pl.ds(sid * RPT, RPT)], acc_sh.at[pl.ds(sid * RPT, RPT)])
    pltpu.sync_copy(ones_hbm, ones_v)
    pltpu.sync_copy(dst_hbm.at[wid], idx_v)
    plsc.subcore_barrier()

    def chunk(c8, carry):
        for j in range(8):
            pltpu.async_copy(ones_v, acc_sh.at[idx_v.at[c8 * 8 + j]], sem, add=True)
        for j in range(8):
            pltpu.make_async_copy(ones_v, acc_sh.at[idx_v.at[c8 * 8 + j]], sem).wait()
        return carry

    lax.fori_loop(0, NCHUNK // 8, chunk, 0)
    plsc.subcore_barrier()
    pltpu.sync_copy(acc_sh.at[pl.ds(sid * RPT, RPT)],
                    out_hbm.at[pl.ds(cid * NP + sid * RPT, RPT)])


@functools.cache
def _deg_call():
  return pl.kernel(
    _deg_body,
    out_type=jax.ShapeDtypeStruct((NC * NP, D), jnp.float32),
    mesh=_sc_mesh(),
    scratch_types=[
        pltpu.VMEM((NCHUNK, CH), jnp.int32),
        pltpu.VMEM((CH, D), jnp.float32),
        pltpu.VMEM_SHARED((NP, D), jnp.float32),
        pltpu.SemaphoreType.DMA,
    ],
  )


def _spmv_body(hn_hbm, src_hbm, dst_hbm, z_hbm, out_hbm,
               isrc0, isrc1, isrc2, isrc3, idst0, idst1, idst2, idst3,
               rows0, rows1, rows2, rows3, acc_sh,
               sg0, sg1, sg2, sg3, ss0, ss1, ss2, ss3,
               sis0, sis1, sis2, sis3, sid0, sid1, sid2, sid3):
    cid = lax.axis_index("c")
    sid = lax.axis_index("s")
    wid = sid * NC + cid
    base = wid * EPW
    rows = (rows0, rows1, rows2, rows3)
    isrc = (isrc0, isrc1, isrc2, isrc3)
    idst = (idst0, idst1, idst2, idst3)
    sg = (sg0, sg1, sg2, sg3)
    ss = (ss0, ss1, ss2, ss3)
    sis = (sis0, sis1, sis2, sis3)
    sidm = (sid0, sid1, sid2, sid3)
    pltpu.sync_copy(z_hbm.at[pl.ds(sid * RPT, RPT)], acc_sh.at[pl.ds(sid * RPT, RPT)])
    plsc.subcore_barrier()

    for j in (0, 1):
        pltpu.sync_copy(src_hbm.at[pl.ds(base + j * CH, CH)], isrc[j])
        pltpu.sync_copy(dst_hbm.at[pl.ds(base + j * CH, CH)], idst[j])
        pltpu.async_copy(hn_hbm.at[isrc[j]], rows[j], sg[j])

    def block(c2, carry):
        for j in range(4):
            c = 4 * c2 + j
            b = j
            b2 = (j + 2) % 4
            if j < 2:
                nxt = c + 2
            else:
                nxt = jnp.where(c2 == NCHUNK // 4 - 1, j - 2, c + 2)
            pltpu.async_copy(src_hbm.at[pl.ds(base + nxt * CH, CH)], isrc[b2], sis[b2])
            pltpu.make_async_copy(hn_hbm.at[isrc[b]], rows[b], sg[b]).wait()
            if j < 2:
                @pl.when(c2 > 0)
                def _():
                    pltpu.make_async_copy(dst_hbm.at[pl.ds(base, CH)], idst[b], sidm[b]).wait()
            else:
                pltpu.make_async_copy(dst_hbm.at[pl.ds(base, CH)], idst[b], sidm[b]).wait()
            pltpu.async_copy(rows[b], acc_sh.at[idst[b]], ss[b], add=True)
            if j < 2:
                @pl.when(c2 > 0)
                def _():
                    pltpu.make_async_copy(rows[b2], acc_sh.at[idst[b2]], ss[b2]).wait()
            else:
                pltpu.make_async_copy(rows[b2], acc_sh.at[idst[b2]], ss[b2]).wait()
            pltpu.async_copy(dst_hbm.at[pl.ds(base + nxt * CH, CH)], idst[b2], sidm[b2])
            pltpu.make_async_copy(src_hbm.at[pl.ds(base, CH)], isrc[b2], sis[b2]).wait()
            pltpu.async_copy(hn_hbm.at[isrc[b2]], rows[b2], sg[b2])
        return carry

    lax.fori_loop(0, NCHUNK // 4, block, 0)
    for j in (0, 1):
        pltpu.make_async_copy(hn_hbm.at[isrc[j]], rows[j], sg[j]).wait()
        pltpu.make_async_copy(dst_hbm.at[pl.ds(base, CH)], idst[j], sidm[j]).wait()
    pltpu.make_async_copy(rows[2], acc_sh.at[idst[2]], ss[2]).wait()
    pltpu.make_async_copy(rows[3], acc_sh.at[idst[3]], ss[3]).wait()
    plsc.subcore_barrier()
    pltpu.sync_copy(acc_sh.at[pl.ds(sid * RPT, RPT)],
                    out_hbm.at[pl.ds(cid * NP + sid * RPT, RPT)])


@functools.cache
def _spmv_call():
  return pl.kernel(
    _spmv_body,
    out_type=jax.ShapeDtypeStruct((NC * NP, D), jnp.float32),
    mesh=_sc_mesh(),
    scratch_types=(
        [pltpu.VMEM((CH,), jnp.int32) for _ in range(8)]
        + [pltpu.VMEM((CH, D), jnp.float32) for _ in range(4)]
        + [pltpu.VMEM_SHARED((NP, D), jnp.float32)]
        + [pltpu.SemaphoreType.DMA for _ in range(16)]
    ),
  )


PPW = B // NW


def _pairs_body(h0, h1, h2, h3, u_hbm, i_hbm,
                ou0, ou1, ou2, ou3, oi0, oi1, oi2, oi3,
                uix, iix, buf, sem):
    cid = lax.axis_index("c")
    sid = lax.axis_index("s")
    wid = sid * NC + cid
    base = wid * PPW
    pltpu.sync_copy(u_hbm.at[pl.ds(base, PPW)], uix)
    pltpu.sync_copy(i_hbm.at[pl.ds(base, PPW)], iix)
    for tbl, ou, oi in ((h0, ou0, oi0), (h1, ou1, oi1),
                        (h2, ou2, oi2), (h3, ou3, oi3)):
        pltpu.async_copy(tbl.at[uix], buf, sem).wait()
        pltpu.sync_copy(buf, ou.at[pl.ds(base, PPW)])
        pltpu.async_copy(tbl.at[iix], buf, sem).wait()
        pltpu.sync_copy(buf, oi.at[pl.ds(base, PPW)])


@functools.cache
def _pairs_call():
  return pl.kernel(
    _pairs_body,
    out_type=tuple(jax.ShapeDtypeStruct((B, D), jnp.float32) for _ in range(8)),
    mesh=_sc_mesh(),
    scratch_types=[
        pltpu.VMEM((PPW,), jnp.int32),
        pltpu.VMEM((PPW,), jnp.int32),
        pltpu.VMEM((PPW, D), jnp.float32),
        pltpu.SemaphoreType.DMA,
    ],
  )


def _prep_body(degp_ref, h0_ref, dinv_ref, hn_ref):
    deg = degp_ref[0:NP, 0:1] + degp_ref[NP:2 * NP, 0:1]
    dinv = lax.rsqrt(jnp.maximum(deg, 1.0))
    dinv_ref[...] = dinv
    hn_ref[...] = h0_ref[...] * dinv


def _tc_prep(degp, h0p):
    return pl.pallas_call(
        _prep_body,
        out_shape=(jax.ShapeDtypeStruct((NP, 1), jnp.float32),
                   jax.ShapeDtypeStruct((NP, D), jnp.float32)),
    )(degp, h0p)


RB = 1264


def _layer_body(h_ref, s0_ref, s1_ref, dinv_ref, w1_ref, w2_ref,
                hp_ref, hn_ref):
    dinv = dinv_ref[...]
    s = dinv * (s0_ref[...] + s1_ref[...])
    h = h_ref[...]
    a = (jnp.dot(h + s, w1_ref[...], preferred_element_type=jnp.float32)
         + jnp.dot(h * s, w2_ref[...], preferred_element_type=jnp.float32))
    hp = jnp.where(a > 0, a, NEG_SLOPE * a)
    hp_ref[...] = hp
    hn_ref[...] = dinv * hp


def _tc_layer(h, sraw, dinv, W1, W2):
    row = lambda i: (i, 0)
    nb = NP // RB
    return pl.pallas_call(
        _layer_body,
        grid=(nb,),
        in_specs=[
            pl.BlockSpec((RB, D), row),
            pl.BlockSpec((RB, D), row),
            pl.BlockSpec((RB, D), lambda i, nb=nb: (i + nb, 0)),
            pl.BlockSpec((RB, 1), row),
            pl.BlockSpec((D, D), lambda i: (0, 0)),
            pl.BlockSpec((D, D), lambda i: (0, 0)),
        ],
        out_specs=(pl.BlockSpec((RB, D), row), pl.BlockSpec((RB, D), row)),
        out_shape=(jax.ShapeDtypeStruct((NP, D), jnp.float32),
                   jax.ShapeDtypeStruct((NP, D), jnp.float32)),
    )(h, sraw, sraw, dinv, W1, W2)


DB = 512


def _dot_body(u0, u1, u2, u3, i0, i1, i2, i3, out_ref):
    acc = jnp.sum(u0[...] * i0[...], axis=1, keepdims=True)
    acc += jnp.sum(u1[...] * i1[...], axis=1, keepdims=True)
    acc += jnp.sum(u2[...] * i2[...], axis=1, keepdims=True)
    acc += jnp.sum(u3[...] * i3[...], axis=1, keepdims=True)
    out_ref[...] = acc


def _tc_dot(gathered):
    row = lambda i: (i, 0)
    return pl.pallas_call(
        _dot_body,
        grid=(B // DB,),
        in_specs=[pl.BlockSpec((DB, D), row) for _ in range(8)],
        out_specs=pl.BlockSpec((DB, 1), row),
        out_shape=jax.ShapeDtypeStruct((B, 1), jnp.float32),
    )(*gathered)


@jax.jit
def kernel(features, feature_values, edge_index, h0,
           W1_0, W2_0, W1_1, W2_1, W1_2, W2_2):
    del feature_values
    src = edge_index[0].astype(jnp.int32)
    dst = edge_index[1].astype(jnp.int32)
    pad = EP - E
    pad_dst = N + (jnp.arange(pad, dtype=jnp.int32) % (NP - N))
    pad_src = jnp.arange(pad, dtype=jnp.int32) % N
    srcp = jnp.concatenate([src, pad_src])
    dstp = jnp.concatenate([dst, pad_dst])
    dst3 = dstp.reshape(NW, NCHUNK, CH)
    zD = jnp.zeros((NP, D), jnp.float32)
    ones = jnp.ones((CH, D), jnp.float32)
    h0p = jnp.pad(h0, ((0, NP - N), (0, 0)))

    degp = _deg_call()(dst3, ones, zD)
    dinv, hn = _tc_prep(degp, h0p)

    h = h0p
    hs = [h0p]
    for (W1, W2) in ((W1_0, W2_0), (W1_1, W2_1), (W1_2, W2_2)):
        sraw = _spmv_call()(hn, srcp, dstp, zD)
        h, hn = _tc_layer(h, sraw, dinv, W1, W2)
        hs.append(h)

    users = features[:, 0].astype(jnp.int32)
    items = features[:, 1].astype(jnp.int32)
    gathered = _pairs_call()(hs[0], hs[1], hs[2], hs[3], users, items)
    return _tc_dot(gathered)[:, 0]
